# baseline (speedup 1.0000x reference)
import functools
import jax
import jax.numpy as jnp
from jax import lax
from jax.experimental import pallas as pl
from jax.experimental.pallas import tpu as pltpu
from jax.experimental.pallas import tpu_sc as plsc

B, N, M, D = 16, 2048, 2048, 3
KP = 16
BS = 4
BT = B - BS
NC, NS, L = 2, 16, 16
NW = NC * NS
G = NW // BS
BPC = BS // NC
HN = N // G
RG = HN // L
CH = M // L
QR = 4

_mesh = plsc.VectorSubcoreMesh(core_axis_name="c", subcore_axis_name="s")


def _tc_body(pred_ref, tgtt_ref, rowsum_ref, colsum_ref):
    x = pred_ref[0]
    yt = tgtt_ref[0]
    xx = jnp.sum(x * x, axis=1, keepdims=True)
    yy = jnp.sum(yt * yt, axis=0, keepdims=True)

    t = jax.lax.dot_general(
        (x * -2.0).astype(jnp.bfloat16), yt.astype(jnp.bfloat16),
        dimension_numbers=(((1,), (0,)), ((), ())),
        preferred_element_type=jnp.float32,
    )

    rmin = jnp.min(t + yy, axis=1, keepdims=True)
    rowsum_ref[0, 0, 0] = jnp.sum(jnp.maximum(rmin + xx, 0.0))
    cmin = jnp.min(t + xx, axis=0, keepdims=True)
    colsum_ref[0, 0, 0] = jnp.sum(jnp.maximum(cmin + yy, 0.0))


def _tc_chamfer(predictions, targets):
    tgt_t = jnp.transpose(targets, (0, 2, 1))
    return pl.pallas_call(
        _tc_body,
        grid=(BT,),
        in_specs=[
            pl.BlockSpec((1, N, D), lambda b: (b, 0, 0)),
            pl.BlockSpec((1, D, M), lambda b: (b, 0, 0)),
        ],
        out_specs=[
            pl.BlockSpec((1, 1, 1), lambda b: (b, 0, 0),
                         memory_space=pltpu.SMEM),
            pl.BlockSpec((1, 1, 1), lambda b: (b, 0, 0),
                         memory_space=pltpu.SMEM),
        ],
        out_shape=[
            jax.ShapeDtypeStruct((BT, 1, 1), jnp.float32),
            jax.ShapeDtypeStruct((BT, 1, 1), jnp.float32),
        ],
        compiler_params=pltpu.CompilerParams(
            dimension_semantics=("parallel",),
        ),
    )(predictions, tgt_t)


@functools.partial(
    pl.kernel, mesh=_mesh,
    out_type=[
        jax.ShapeDtypeStruct((NW, L), jnp.float32),
        jax.ShapeDtypeStruct((NW, L), jnp.float32),
    ],
    scratch_types=[
        pltpu.VMEM((D, HN), jnp.float32),
        pltpu.VMEM((HN,), jnp.float32),
        pltpu.VMEM((D, M), jnp.float32),
        pltpu.VMEM((M,), jnp.float32),
        pltpu.VMEM((M,), jnp.float32),
        pltpu.VMEM((M,), jnp.float32),
        pltpu.VMEM((L, L), jnp.float32),
        pltpu.VMEM((L,), jnp.float32),
        pltpu.VMEM((L,), jnp.float32),
        pltpu.VMEM_SHARED((NS, M), jnp.float32),
    ],
    compiler_params=pltpu.CompilerParams(needs_layout_passes=False),
)
def _sc_chamfer(xm2_hbm, xx_hbm, ytb_hbm, yy_hbm, row_out, col_out,
                xrow_v, xx_v, y_v, yy_v, cmin_v, cnb_v, tr_v,
                rpart_v, cpart_v, shared):
    c = lax.axis_index("c")
    s = lax.axis_index("s")
    w = c * NS + s
    b = c * BPC + s // G
    part = s % G
    off = part * HN

    pltpu.sync_copy(xm2_hbm.at[b, :, pl.ds(off, HN)], xrow_v)
    pltpu.sync_copy(xx_hbm.at[b, pl.ds(off, HN)], xx_v)
    pltpu.sync_copy(ytb_hbm.at[b], y_v)
    pltpu.sync_copy(yy_hbm.at[b], yy_v)

    inf = jnp.float32(jnp.inf)
    inf_vec = jnp.full((L,), inf, jnp.float32)

    def _init_cmin(j, _):
        cmin_v[pl.ds(j * L, L)] = inf_vec
        return 0
    lax.fori_loop(0, CH, _init_cmin, 0)

    def _rowgroup(rg, rowsum_acc):
        xv0 = xrow_v[0, pl.ds(rg * L, L)]
        xv1 = xrow_v[1, pl.ds(rg * L, L)]
        xv2 = xrow_v[2, pl.ds(rg * L, L)]
        xxv = xx_v[pl.ds(rg * L, L)]

        for q in range(L // QR):
            xb = [(jnp.full((L,), xv0[q * QR + r], jnp.float32),
                   jnp.full((L,), xv1[q * QR + r], jnp.float32),
                   jnp.full((L,), xv2[q * QR + r], jnp.float32),
                   jnp.full((L,), xxv[q * QR + r], jnp.float32))
                  for r in range(QR)]

            @plsc.parallel_loop(0, CH, carry=(inf_vec,) * QR, unroll=2)
            def _chunk(j, rmins):
                y0 = y_v[0, pl.ds(j * L, L)]
                y1 = y_v[1, pl.ds(j * L, L)]
                y2 = y_v[2, pl.ds(j * L, L)]
                yyc = yy_v[pl.ds(j * L, L)]
                new_rmins = []
                vs = []
                for r in range(QR):
                    b0, b1, b2, bxx = xb[r]
                    t = (b0 * y0 + b1 * y1) + b2 * y2
                    new_rmins.append(jnp.minimum(rmins[r], t + yyc))
                    vs.append(t + bxx)
                while len(vs) > 1:
                    vs = [jnp.minimum(vs[i], vs[i + 1])
                          for i in range(0, len(vs) - 1, 2)] \
                         + ([vs[-1]] if len(vs) % 2 else [])
                vq = vs[0]
                cmin_v[pl.ds(j * L, L)] = jnp.minimum(
                    cmin_v[pl.ds(j * L, L)], vq)
                return tuple(new_rmins)

            rmins = _chunk
            for r in range(QR):
                tr_v[q * QR + r] = rmins[r]

        iot = lax.iota(jnp.int32, L)
        rmv = inf_vec
        for j in range(L):
            col = plsc.load_gather(tr_v, [iot, jnp.full((L,), j, jnp.int32)])
            rmv = jnp.minimum(rmv, col)
        return rowsum_acc + jnp.maximum(rmv + xxv, 0.0)

    rowsum = lax.fori_loop(0, RG, _rowgroup, jnp.zeros((L,), jnp.float32))

    rpart_v[...] = rowsum
    pltpu.sync_copy(rpart_v, row_out.at[w])

    pltpu.sync_copy(cmin_v, shared.at[s])
    plsc.subcore_barrier()

    cpart_v[...] = jnp.zeros((L,), jnp.float32)

    @pl.when(part == 0)
    def _leader():
        for p in range(1, G):
            pltpu.sync_copy(shared.at[s + p], cnb_v)

            def _mrg(j, _):
                cmin_v[pl.ds(j * L, L)] = jnp.minimum(
                    cmin_v[pl.ds(j * L, L)], cnb_v[pl.ds(j * L, L)])
                return 0
            lax.fori_loop(0, CH, _mrg, 0)

        def _colfin(j, acc):
            cm = cmin_v[pl.ds(j * L, L)] + yy_v[pl.ds(j * L, L)]
            return acc + jnp.maximum(cm, 0.0)
        colsum = lax.fori_loop(0, CH, _colfin, jnp.zeros((L,), jnp.float32))
        cpart_v[...] = colsum

    pltpu.sync_copy(cpart_v, col_out.at[w])


def _round_bf16(x):
    b = jax.lax.bitcast_convert_type(x, jnp.uint32)
    r = (b + jnp.uint32(0x7FFF) + ((b >> 16) & jnp.uint32(1))) \
        & jnp.uint32(0xFFFF0000)
    return jax.lax.bitcast_convert_type(r, jnp.float32)


def kernel(predictions, targets):
    pred_sc, pred_tc = predictions[:BS], predictions[BS:]
    tgt_sc, tgt_tc = targets[:BS], targets[BS:]

    xm2t = jnp.transpose(_round_bf16(pred_sc * -2.0), (0, 2, 1))
    ytbt = jnp.transpose(_round_bf16(tgt_sc), (0, 2, 1))
    xx = jnp.sum(pred_sc * pred_sc, axis=2)
    yy = jnp.sum(tgt_sc * tgt_sc, axis=2)
    row_sc, col_sc = _sc_chamfer(xm2t, xx, ytbt, yy)

    rowsum_tc, colsum_tc = _tc_chamfer(pred_tc, tgt_tc)

    rowtot = jnp.sum(rowsum_tc) + jnp.sum(row_sc)
    coltot = jnp.sum(colsum_tc) + jnp.sum(col_sc)
    return rowtot / (B * N) + coltot / (B * M)


# speedup vs baseline: 1.1773x; 1.1773x over previous
import functools
import jax
import jax.numpy as jnp
from jax import lax
from jax.experimental import pallas as pl
from jax.experimental.pallas import tpu as pltpu
from jax.experimental.pallas import tpu_sc as plsc

B, N, M, D = 16, 2048, 2048, 3
KP = 16
BS = 2
BT = B - BS
NC, NS, L = 2, 16, 16
NW = NC * NS
G = NW // BS
BPC = BS // NC
HN = N // G
RG = HN // L
CH = M // L
QR = 4

_mesh = plsc.VectorSubcoreMesh(core_axis_name="c", subcore_axis_name="s")


def _tc_body(pred_ref, tgtt_ref, rowsum_ref, colsum_ref):
    x = pred_ref[0]
    yt = tgtt_ref[0]
    xx = jnp.sum(x * x, axis=1, keepdims=True)
    yy = jnp.sum(yt * yt, axis=0, keepdims=True)

    t = jax.lax.dot_general(
        (x * -2.0).astype(jnp.bfloat16), yt.astype(jnp.bfloat16),
        dimension_numbers=(((1,), (0,)), ((), ())),
        preferred_element_type=jnp.float32,
    )

    rmin = jnp.min(t + yy, axis=1, keepdims=True)
    rowsum_ref[0, 0, 0] = jnp.sum(jnp.maximum(rmin + xx, 0.0))
    cmin = jnp.min(t + xx, axis=0, keepdims=True)
    colsum_ref[0, 0, 0] = jnp.sum(jnp.maximum(cmin + yy, 0.0))


def _tc_chamfer(predictions, targets):
    tgt_t = jnp.transpose(targets, (0, 2, 1))
    return pl.pallas_call(
        _tc_body,
        grid=(BT,),
        in_specs=[
            pl.BlockSpec((1, N, D), lambda b: (b, 0, 0)),
            pl.BlockSpec((1, D, M), lambda b: (b, 0, 0)),
        ],
        out_specs=[
            pl.BlockSpec((1, 1, 1), lambda b: (b, 0, 0),
                         memory_space=pltpu.SMEM),
            pl.BlockSpec((1, 1, 1), lambda b: (b, 0, 0),
                         memory_space=pltpu.SMEM),
        ],
        out_shape=[
            jax.ShapeDtypeStruct((BT, 1, 1), jnp.float32),
            jax.ShapeDtypeStruct((BT, 1, 1), jnp.float32),
        ],
        compiler_params=pltpu.CompilerParams(
            dimension_semantics=("parallel",),
        ),
    )(predictions, tgt_t)


@functools.partial(
    pl.kernel, mesh=_mesh,
    out_type=[
        jax.ShapeDtypeStruct((NW, L), jnp.float32),
        jax.ShapeDtypeStruct((NW, L), jnp.float32),
    ],
    scratch_types=[
        pltpu.VMEM((D, HN), jnp.float32),
        pltpu.VMEM((HN,), jnp.float32),
        pltpu.VMEM((D, M), jnp.float32),
        pltpu.VMEM((M,), jnp.float32),
        pltpu.VMEM((M,), jnp.float32),
        pltpu.VMEM((M,), jnp.float32),
        pltpu.VMEM((L, L), jnp.float32),
        pltpu.VMEM((L,), jnp.float32),
        pltpu.VMEM((L,), jnp.float32),
        pltpu.VMEM_SHARED((NS, M), jnp.float32),
    ],
    compiler_params=pltpu.CompilerParams(needs_layout_passes=False),
)
def _sc_chamfer(xm2_hbm, xx_hbm, ytb_hbm, yy_hbm, row_out, col_out,
                xrow_v, xx_v, y_v, yy_v, cmin_v, cnb_v, tr_v,
                rpart_v, cpart_v, shared):
    c = lax.axis_index("c")
    s = lax.axis_index("s")
    w = c * NS + s
    b = c * BPC + s // G
    part = s % G
    off = part * HN

    pltpu.sync_copy(xm2_hbm.at[b, :, pl.ds(off, HN)], xrow_v)
    pltpu.sync_copy(xx_hbm.at[b, pl.ds(off, HN)], xx_v)
    pltpu.sync_copy(ytb_hbm.at[b], y_v)
    pltpu.sync_copy(yy_hbm.at[b], yy_v)

    inf = jnp.float32(jnp.inf)
    inf_vec = jnp.full((L,), inf, jnp.float32)

    def _init_cmin(j, _):
        cmin_v[pl.ds(j * L, L)] = inf_vec
        return 0
    lax.fori_loop(0, CH, _init_cmin, 0)

    def _rowgroup(rg, rowsum_acc):
        xv0 = xrow_v[0, pl.ds(rg * L, L)]
        xv1 = xrow_v[1, pl.ds(rg * L, L)]
        xv2 = xrow_v[2, pl.ds(rg * L, L)]
        xxv = xx_v[pl.ds(rg * L, L)]

        for q in range(L // QR):
            xb = [(jnp.full((L,), xv0[q * QR + r], jnp.float32),
                   jnp.full((L,), xv1[q * QR + r], jnp.float32),
                   jnp.full((L,), xv2[q * QR + r], jnp.float32),
                   jnp.full((L,), xxv[q * QR + r], jnp.float32))
                  for r in range(QR)]

            @plsc.parallel_loop(0, CH, carry=(inf_vec,) * QR, unroll=2)
            def _chunk(j, rmins):
                y0 = y_v[0, pl.ds(j * L, L)]
                y1 = y_v[1, pl.ds(j * L, L)]
                y2 = y_v[2, pl.ds(j * L, L)]
                yyc = yy_v[pl.ds(j * L, L)]
                new_rmins = []
                vs = []
                for r in range(QR):
                    b0, b1, b2, bxx = xb[r]
                    t = (b0 * y0 + b1 * y1) + b2 * y2
                    new_rmins.append(jnp.minimum(rmins[r], t + yyc))
                    vs.append(t + bxx)
                while len(vs) > 1:
                    vs = [jnp.minimum(vs[i], vs[i + 1])
                          for i in range(0, len(vs) - 1, 2)] \
                         + ([vs[-1]] if len(vs) % 2 else [])
                vq = vs[0]
                cmin_v[pl.ds(j * L, L)] = jnp.minimum(
                    cmin_v[pl.ds(j * L, L)], vq)
                return tuple(new_rmins)

            rmins = _chunk
            for r in range(QR):
                tr_v[q * QR + r] = rmins[r]

        iot = lax.iota(jnp.int32, L)
        rmv = inf_vec
        for j in range(L):
            col = plsc.load_gather(tr_v, [iot, jnp.full((L,), j, jnp.int32)])
            rmv = jnp.minimum(rmv, col)
        return rowsum_acc + jnp.maximum(rmv + xxv, 0.0)

    rowsum = lax.fori_loop(0, RG, _rowgroup, jnp.zeros((L,), jnp.float32))

    rpart_v[...] = rowsum
    pltpu.sync_copy(rpart_v, row_out.at[w])

    pltpu.sync_copy(cmin_v, shared.at[s])
    plsc.subcore_barrier()

    cpart_v[...] = jnp.zeros((L,), jnp.float32)

    @pl.when(part == 0)
    def _leader():
        for p in range(1, G):
            pltpu.sync_copy(shared.at[s + p], cnb_v)

            def _mrg(j, _):
                cmin_v[pl.ds(j * L, L)] = jnp.minimum(
                    cmin_v[pl.ds(j * L, L)], cnb_v[pl.ds(j * L, L)])
                return 0
            lax.fori_loop(0, CH, _mrg, 0)

        def _colfin(j, acc):
            cm = cmin_v[pl.ds(j * L, L)] + yy_v[pl.ds(j * L, L)]
            return acc + jnp.maximum(cm, 0.0)
        colsum = lax.fori_loop(0, CH, _colfin, jnp.zeros((L,), jnp.float32))
        cpart_v[...] = colsum

    pltpu.sync_copy(cpart_v, col_out.at[w])


def _round_bf16(x):
    b = jax.lax.bitcast_convert_type(x, jnp.uint32)
    r = (b + jnp.uint32(0x7FFF) + ((b >> 16) & jnp.uint32(1))) \
        & jnp.uint32(0xFFFF0000)
    return jax.lax.bitcast_convert_type(r, jnp.float32)


def kernel(predictions, targets):
    pred_sc, pred_tc = predictions[:BS], predictions[BS:]
    tgt_sc, tgt_tc = targets[:BS], targets[BS:]

    xm2t = jnp.transpose(_round_bf16(pred_sc * -2.0), (0, 2, 1))
    ytbt = jnp.transpose(_round_bf16(tgt_sc), (0, 2, 1))
    xx = jnp.sum(pred_sc * pred_sc, axis=2)
    yy = jnp.sum(tgt_sc * tgt_sc, axis=2)
    row_sc, col_sc = _sc_chamfer(xm2t, xx, ytbt, yy)

    rowsum_tc, colsum_tc = _tc_chamfer(pred_tc, tgt_tc)

    rowtot = jnp.sum(rowsum_tc) + jnp.sum(row_sc)
    coltot = jnp.sum(colsum_tc) + jnp.sum(col_sc)
    return rowtot / (B * N) + coltot / (B * M)


# speedup vs baseline: 1.1987x; 1.0182x over previous
import functools
import jax
import jax.numpy as jnp
from jax import lax
from jax.experimental import pallas as pl
from jax.experimental.pallas import tpu as pltpu
from jax.experimental.pallas import tpu_sc as plsc

B, N, M, D = 16, 2048, 2048, 3
KP = 16
BS = 2
BT = B - BS
NC, NS, L = 2, 16, 16
NW = NC * NS
G = NW // BS
BPC = BS // NC
HN = N // G
RG = HN // L
CH = M // L
QR = 4

_mesh = plsc.VectorSubcoreMesh(core_axis_name="c", subcore_axis_name="s")


def _tc_body(pred_ref, tgtt_ref, rowsum_ref, colsum_ref):
    x = pred_ref[0]
    yt = tgtt_ref[0]
    xx = jnp.sum(x * x, axis=1, keepdims=True)
    yy = jnp.sum(yt * yt, axis=0, keepdims=True)

    t = jax.lax.dot_general(
        (x * -2.0).astype(jnp.bfloat16), yt.astype(jnp.bfloat16),
        dimension_numbers=(((1,), (0,)), ((), ())),
        preferred_element_type=jnp.float32,
    )

    rmin = jnp.min(t + yy, axis=1, keepdims=True)
    rowsum_ref[0, 0, 0] = jnp.sum(jnp.maximum(rmin + xx, 0.0))
    cmin = jnp.min(t + xx, axis=0, keepdims=True)
    colsum_ref[0, 0, 0] = jnp.sum(jnp.maximum(cmin + yy, 0.0))


def _tc_chamfer(predictions, targets):
    tgt_t = jnp.transpose(targets, (0, 2, 1))
    return pl.pallas_call(
        _tc_body,
        grid=(BT,),
        in_specs=[
            pl.BlockSpec((1, N, D), lambda b: (b, 0, 0)),
            pl.BlockSpec((1, D, M), lambda b: (b, 0, 0)),
        ],
        out_specs=[
            pl.BlockSpec((1, 1, 1), lambda b: (b, 0, 0),
                         memory_space=pltpu.SMEM),
            pl.BlockSpec((1, 1, 1), lambda b: (b, 0, 0),
                         memory_space=pltpu.SMEM),
        ],
        out_shape=[
            jax.ShapeDtypeStruct((BT, 1, 1), jnp.float32),
            jax.ShapeDtypeStruct((BT, 1, 1), jnp.float32),
        ],
        compiler_params=pltpu.CompilerParams(
            dimension_semantics=("parallel",),
            allow_input_fusion=(True, True),
        ),
    )(predictions, tgt_t)


@functools.partial(
    pl.kernel, mesh=_mesh,
    out_type=[
        jax.ShapeDtypeStruct((NW, L), jnp.float32),
        jax.ShapeDtypeStruct((NW, L), jnp.float32),
    ],
    scratch_types=[
        pltpu.VMEM((D, HN), jnp.float32),
        pltpu.VMEM((HN,), jnp.float32),
        pltpu.VMEM((D, M), jnp.float32),
        pltpu.VMEM((M,), jnp.float32),
        pltpu.VMEM((M,), jnp.float32),
        pltpu.VMEM((M,), jnp.float32),
        pltpu.VMEM((L, L), jnp.float32),
        pltpu.VMEM((L,), jnp.float32),
        pltpu.VMEM((L,), jnp.float32),
        pltpu.VMEM_SHARED((NS, M), jnp.float32),
    ],
    compiler_params=pltpu.CompilerParams(needs_layout_passes=False),
)
def _sc_chamfer(xm2_hbm, xx_hbm, ytb_hbm, yy_hbm, row_out, col_out,
                xrow_v, xx_v, y_v, yy_v, cmin_v, cnb_v, tr_v,
                rpart_v, cpart_v, shared):
    c = lax.axis_index("c")
    s = lax.axis_index("s")
    w = c * NS + s
    b = c * BPC + s // G
    part = s % G
    off = part * HN

    pltpu.sync_copy(xm2_hbm.at[b, :, pl.ds(off, HN)], xrow_v)
    pltpu.sync_copy(xx_hbm.at[b, pl.ds(off, HN)], xx_v)
    pltpu.sync_copy(ytb_hbm.at[b], y_v)
    pltpu.sync_copy(yy_hbm.at[b], yy_v)

    inf = jnp.float32(jnp.inf)
    inf_vec = jnp.full((L,), inf, jnp.float32)

    def _init_cmin(j, _):
        cmin_v[pl.ds(j * L, L)] = inf_vec
        return 0
    lax.fori_loop(0, CH, _init_cmin, 0)

    def _rowgroup(rg, rowsum_acc):
        xv0 = xrow_v[0, pl.ds(rg * L, L)]
        xv1 = xrow_v[1, pl.ds(rg * L, L)]
        xv2 = xrow_v[2, pl.ds(rg * L, L)]
        xxv = xx_v[pl.ds(rg * L, L)]

        for q in range(L // QR):
            xb = [(jnp.full((L,), xv0[q * QR + r], jnp.float32),
                   jnp.full((L,), xv1[q * QR + r], jnp.float32),
                   jnp.full((L,), xv2[q * QR + r], jnp.float32),
                   jnp.full((L,), xxv[q * QR + r], jnp.float32))
                  for r in range(QR)]

            @plsc.parallel_loop(0, CH, carry=(inf_vec,) * QR, unroll=2)
            def _chunk(j, rmins):
                y0 = y_v[0, pl.ds(j * L, L)]
                y1 = y_v[1, pl.ds(j * L, L)]
                y2 = y_v[2, pl.ds(j * L, L)]
                yyc = yy_v[pl.ds(j * L, L)]
                new_rmins = []
                vs = []
                for r in range(QR):
                    b0, b1, b2, bxx = xb[r]
                    t = (b0 * y0 + b1 * y1) + b2 * y2
                    new_rmins.append(jnp.minimum(rmins[r], t + yyc))
                    vs.append(t + bxx)
                while len(vs) > 1:
                    vs = [jnp.minimum(vs[i], vs[i + 1])
                          for i in range(0, len(vs) - 1, 2)] \
                         + ([vs[-1]] if len(vs) % 2 else [])
                vq = vs[0]
                cmin_v[pl.ds(j * L, L)] = jnp.minimum(
                    cmin_v[pl.ds(j * L, L)], vq)
                return tuple(new_rmins)

            rmins = _chunk
            for r in range(QR):
                tr_v[q * QR + r] = rmins[r]

        iot = lax.iota(jnp.int32, L)
        rmv = inf_vec
        for j in range(L):
            col = plsc.load_gather(tr_v, [iot, jnp.full((L,), j, jnp.int32)])
            rmv = jnp.minimum(rmv, col)
        return rowsum_acc + jnp.maximum(rmv + xxv, 0.0)

    rowsum = lax.fori_loop(0, RG, _rowgroup, jnp.zeros((L,), jnp.float32))

    rpart_v[...] = rowsum
    pltpu.sync_copy(rpart_v, row_out.at[w])

    pltpu.sync_copy(cmin_v, shared.at[s])
    plsc.subcore_barrier()

    cpart_v[...] = jnp.zeros((L,), jnp.float32)

    @pl.when(part == 0)
    def _leader():
        for p in range(1, G):
            pltpu.sync_copy(shared.at[s + p], cnb_v)

            def _mrg(j, _):
                cmin_v[pl.ds(j * L, L)] = jnp.minimum(
                    cmin_v[pl.ds(j * L, L)], cnb_v[pl.ds(j * L, L)])
                return 0
            lax.fori_loop(0, CH, _mrg, 0)

        def _colfin(j, acc):
            cm = cmin_v[pl.ds(j * L, L)] + yy_v[pl.ds(j * L, L)]
            return acc + jnp.maximum(cm, 0.0)
        colsum = lax.fori_loop(0, CH, _colfin, jnp.zeros((L,), jnp.float32))
        cpart_v[...] = colsum

    pltpu.sync_copy(cpart_v, col_out.at[w])


def _round_bf16(x):
    b = jax.lax.bitcast_convert_type(x, jnp.uint32)
    r = (b + jnp.uint32(0x7FFF) + ((b >> 16) & jnp.uint32(1))) \
        & jnp.uint32(0xFFFF0000)
    return jax.lax.bitcast_convert_type(r, jnp.float32)


def kernel(predictions, targets):
    pred_sc, pred_tc = predictions[:BS], predictions[BS:]
    tgt_sc, tgt_tc = targets[:BS], targets[BS:]

    xm2t = jnp.transpose(_round_bf16(pred_sc * -2.0), (0, 2, 1))
    ytbt = jnp.transpose(_round_bf16(tgt_sc), (0, 2, 1))
    xx = jnp.sum(pred_sc * pred_sc, axis=2)
    yy = jnp.sum(tgt_sc * tgt_sc, axis=2)
    row_sc, col_sc = _sc_chamfer(xm2t, xx, ytbt, yy)

    rowsum_tc, colsum_tc = _tc_chamfer(pred_tc, tgt_tc)

    rowtot = jnp.sum(rowsum_tc) + jnp.sum(row_sc)
    coltot = jnp.sum(colsum_tc) + jnp.sum(col_sc)
    return rowtot / (B * N) + coltot / (B * M)
